# baseline (speedup 1.0000x reference)
import functools
import math

import jax
import jax.numpy as jnp
from jax import lax
from jax.experimental import pallas as pl
from jax.experimental.pallas import tpu as pltpu
from jax.experimental.pallas import tpu_sc as plsc

D = 64
SCALE = math.sqrt(D)
L = 16
NC = 2
NS = 16
NW = NC * NS
C = 128


def _make_emb_kernel(B: int):
    assert B % (NW * C) == 0
    b_per_w = B // NW
    n_chunks = b_per_w // C
    mesh = plsc.VectorSubcoreMesh(core_axis_name="c", subcore_axis_name="s")

    @functools.partial(
        pl.kernel,
        out_type=jax.ShapeDtypeStruct((B, D), jnp.float32),
        mesh=mesh,
        compiler_params=pltpu.CompilerParams(use_tc_tiling_on_sc=False),
        scratch_types=[
            pltpu.VMEM((C,), jnp.int32),
            pltpu.VMEM((C, D), jnp.float32),
            pltpu.SemaphoreType.DMA,
        ],
    )
    def emb_kernel(x_hbm, lut_hbm, out_hbm, idx_v, rows_v, sem):
        wid = lax.axis_index("s") * NC + lax.axis_index("c")
        base = wid * b_per_w

        def chunk_body(g, _):
            off = base + g * C
            pltpu.sync_copy(x_hbm.at[pl.ds(off, C)], idx_v)
            pltpu.async_copy(lut_hbm.at[idx_v], rows_v, sem).wait()

            def scale_row(i, _):
                for j in range(D // L):
                    sl = pl.ds(j * L, L)
                    rows_v[i, sl] = rows_v[i, sl] * SCALE
                return 0

            lax.fori_loop(0, C, scale_row, 0)
            pltpu.sync_copy(rows_v, out_hbm.at[pl.ds(off, C)])
            return 0

        lax.fori_loop(0, n_chunks, chunk_body, 0)

    return emb_kernel


def kernel(x, lut):
    B = x.shape[0] * x.shape[1]
    xf = x.reshape(B).astype(jnp.int32)
    out = _make_emb_kernel(B)(xf, lut)
    return out.reshape(x.shape[0], x.shape[1], D)


# speedup vs baseline: 1.2803x; 1.2803x over previous
import functools
import math

import jax
import jax.numpy as jnp
from jax import lax
from jax.experimental import pallas as pl
from jax.experimental.pallas import tpu as pltpu
from jax.experimental.pallas import tpu_sc as plsc

D = 64
SCALE = math.sqrt(D)
L = 16
NC = 2
NS = 16
NW = NC * NS
C = 128
NBUF = 4
UNROLL = 4


def _make_emb_kernel(B: int):
    assert B % (NW * C) == 0
    b_per_w = B // NW
    n_chunks = b_per_w // C
    assert n_chunks % NBUF == 0
    mesh = plsc.VectorSubcoreMesh(core_axis_name="c", subcore_axis_name="s")

    @functools.partial(
        pl.kernel,
        out_type=jax.ShapeDtypeStruct((B, D), jnp.float32),
        mesh=mesh,
        compiler_params=pltpu.CompilerParams(use_tc_tiling_on_sc=False),
        scratch_types=(
            [pltpu.VMEM((n_chunks, C), jnp.int32)]
            + [pltpu.VMEM((C, D), jnp.float32) for _ in range(2 * NBUF)]
            + [pltpu.SemaphoreType.DMA for _ in range(2 * NBUF)]
        ),
    )
    def emb_kernel(x_hbm, lut_hbm, out_hbm, idx_v, *bufs_and_sems):
        rows_in = bufs_and_sems[:NBUF]
        rows_out = bufs_and_sems[NBUF:2 * NBUF]
        gsem = bufs_and_sems[2 * NBUF:3 * NBUF]
        ssem = bufs_and_sems[3 * NBUF:4 * NBUF]

        wid = lax.axis_index("s") * NC + lax.axis_index("c")
        base = wid * n_chunks

        pltpu.sync_copy(x_hbm.at[pl.ds(base, n_chunks)], idx_v)

        for b in range(NBUF):
            pltpu.async_copy(lut_hbm.at[idx_v.at[b]], rows_in[b], gsem[b])

        def step(s, _):
            for b in range(NBUF):
                g = s * NBUF + b
                pltpu.make_async_copy(
                    lut_hbm.at[idx_v.at[g]], rows_in[b], gsem[b]).wait()
                @pl.when(s > 0)
                def _():
                    pltpu.make_async_copy(
                        rows_out[b],
                        out_hbm.at[pl.ds((base + g - NBUF) * C, C)],
                        ssem[b]).wait()

                def scale_rows(i, _):
                    for u in range(UNROLL):
                        r = i * UNROLL + u
                        for j in range(D // L):
                            sl = pl.ds(j * L, L)
                            rows_out[b][r, sl] = rows_in[b][r, sl] * SCALE
                    return 0

                lax.fori_loop(0, C // UNROLL, scale_rows, 0)
                pltpu.async_copy(
                    rows_out[b], out_hbm.at[pl.ds((base + g) * C, C)], ssem[b])

                @pl.when(s < n_chunks // NBUF - 1)
                def _():
                    pltpu.async_copy(
                        lut_hbm.at[idx_v.at[g + NBUF]], rows_in[b], gsem[b])
            return 0

        lax.fori_loop(0, n_chunks // NBUF, step, 0)

        for b in range(NBUF):
            g = n_chunks - NBUF + b
            pltpu.make_async_copy(
                rows_out[b], out_hbm.at[pl.ds((base + g) * C, C)],
                ssem[b]).wait()

    return emb_kernel


def kernel(x, lut):
    B = x.shape[0] * x.shape[1]
    xf = x.reshape(B // C, C).astype(jnp.int32)
    out = _make_emb_kernel(B)(xf, lut)
    return out.reshape(x.shape[0], x.shape[1], D)


# speedup vs baseline: 1.3217x; 1.0324x over previous
import functools
import math

import jax
import jax.numpy as jnp
from jax import lax
from jax.experimental import pallas as pl
from jax.experimental.pallas import tpu as pltpu
from jax.experimental.pallas import tpu_sc as plsc

D = 64
W = 128
SCALE = math.sqrt(D)
L = 16
NC = 2
NS = 16
NW = NC * NS
C = 128
NBUF = 2


def _make_emb_kernel(B: int):
    assert B % (NW * C) == 0
    b_per_w = B // NW
    n_chunks = b_per_w // C
    assert n_chunks % NBUF == 0
    mesh = plsc.VectorSubcoreMesh(core_axis_name="c", subcore_axis_name="s")

    @functools.partial(
        pl.kernel,
        out_type=jax.ShapeDtypeStruct((B, W), jnp.float32),
        mesh=mesh,
        compiler_params=pltpu.CompilerParams(use_tc_tiling_on_sc=False),
        scratch_types=(
            [pltpu.VMEM((b_per_w,), jnp.int32)]
            + [pltpu.VMEM((C, W), jnp.float32) for _ in range(NBUF)]
            + [pltpu.VMEM((C, D), jnp.float32) for _ in range(NBUF)]
            + [pltpu.SemaphoreType.DMA for _ in range(2 * NBUF)]
        ),
    )
    def emb_kernel(x_hbm, lut_hbm, out_hbm, idx_v, *scratch):
        rows_in = scratch[0:NBUF]
        rows_out = scratch[NBUF:2 * NBUF]
        gsem = scratch[2 * NBUF:3 * NBUF]
        ssem = scratch[3 * NBUF:4 * NBUF]

        wid = lax.axis_index("s") * NC + lax.axis_index("c")
        base = wid * b_per_w

        pltpu.sync_copy(x_hbm.at[pl.ds(base, b_per_w)], idx_v)

        def gather_copy(t, b):
            return pltpu.make_async_copy(
                lut_hbm.at[idx_v.at[pl.ds(t * C, C)]], rows_in[b], gsem[b])

        def store_copy(t, b):
            return pltpu.make_async_copy(
                rows_out[b],
                out_hbm.at[pl.ds(base + t * C, C), pl.ds(0, D)], ssem[b])

        for b in range(NBUF):
            gather_copy(b, b).start()

        def step(s, _):
            for b in range(NBUF):
                t = s * NBUF + b
                gather_copy(t, b).wait()

                @pl.when(s > 0)
                def _():
                    store_copy(t - NBUF, b).wait()

                def scale_rows(i, _):
                    for u in range(4):
                        r = i * 4 + u
                        for c in range(D // L):
                            sl = pl.ds(c * L, L)
                            rows_out[b][r, sl] = rows_in[b][r, sl] * SCALE
                    return 0

                lax.fori_loop(0, C // 4, scale_rows, 0)
                store_copy(t, b).start()

                @pl.when(s < n_chunks // NBUF - 1)
                def _():
                    gather_copy(t + NBUF, b).start()
            return 0

        lax.fori_loop(0, n_chunks // NBUF, step, 0)

        for b in range(NBUF):
            store_copy(n_chunks - NBUF + b, b).wait()

    return emb_kernel


def kernel(x, lut):
    NB, T = x.shape
    B = NB * T
    lutp = jnp.concatenate(
        [lut, jnp.zeros((lut.shape[0], W - D), jnp.float32)], axis=1)
    out = _make_emb_kernel(B)(x.reshape(B).astype(jnp.int32), lutp)
    return out[:, :D].reshape(NB, T, D)


# speedup vs baseline: 1.5606x; 1.1807x over previous
import functools
import math

import jax
import jax.numpy as jnp
from jax import lax
from jax.experimental import pallas as pl
from jax.experimental.pallas import tpu as pltpu
from jax.experimental.pallas import tpu_sc as plsc

D = 64
W = 128
SCALE = math.sqrt(D)
L = 16
NC = 2
NS = 16
NW = NC * NS
C = 128
NBUF = 3


def _make_emb_kernel(B: int):
    assert B % (NW * C) == 0
    b_per_w = B // NW
    n_chunks = b_per_w // C
    n_main = (n_chunks // NBUF) - 1
    n_peel = n_chunks - NBUF * n_main
    mesh = plsc.VectorSubcoreMesh(core_axis_name="c", subcore_axis_name="s")

    @functools.partial(
        pl.kernel,
        out_type=jax.ShapeDtypeStruct((B, W), jnp.float32),
        mesh=mesh,
        compiler_params=pltpu.CompilerParams(use_tc_tiling_on_sc=False),
        scratch_types=(
            [pltpu.VMEM((b_per_w,), jnp.int32)]
            + [pltpu.VMEM((C, W), jnp.float32) for _ in range(2 * NBUF)]
            + [pltpu.SemaphoreType.DMA for _ in range(2 * NBUF)]
        ),
    )
    def emb_kernel(x_hbm, lut_hbm, out_hbm, idx_v, *scratch):
        rows_in = scratch[0:NBUF]
        rows_out = scratch[NBUF:2 * NBUF]
        gsem = scratch[2 * NBUF:3 * NBUF]
        ssem = scratch[3 * NBUF:4 * NBUF]

        wid = lax.axis_index("s") * NC + lax.axis_index("c")
        base = wid * b_per_w

        pltpu.sync_copy(x_hbm.at[pl.ds(base, b_per_w)], idx_v)

        def gather_copy(t, b):
            return pltpu.make_async_copy(
                lut_hbm.at[idx_v.at[pl.ds(t * C, C)]], rows_in[b], gsem[b])

        def store_copy(t, b):
            return pltpu.make_async_copy(
                rows_out[b], out_hbm.at[pl.ds(base + t * C, C)], ssem[b])

        def scale_chunk(b):
            def scale_rows(i, _):
                for u in range(4):
                    r = i * 4 + u
                    for c in range(D // L):
                        sl = pl.ds(c * L, L)
                        rows_out[b][r, sl] = rows_in[b][r, sl] * SCALE
                return 0

            lax.fori_loop(0, C // 4, scale_rows, 0)

        for b in range(NBUF):
            gather_copy(b, b).start()

        def step(s, _):
            for b in range(NBUF):
                t = s * NBUF + b
                gather_copy(t, b).wait()

                @pl.when(s > 0)
                def _():
                    store_copy(t - NBUF, b).wait()

                scale_chunk(b)
                store_copy(t, b).start()
                gather_copy(t + NBUF, b).start()
            return 0

        lax.fori_loop(0, n_main, step, 0)

        for p in range(n_peel):
            t = NBUF * n_main + p
            b = t % NBUF
            gather_copy(t, b).wait()
            if t - NBUF >= 0:
                store_copy(t - NBUF, b).wait()
            scale_chunk(b)
            store_copy(t, b).start()
            if t + NBUF < n_chunks:
                gather_copy(t + NBUF, b).start()

        for t in range(n_chunks - NBUF, n_chunks):
            store_copy(t, t % NBUF).wait()

    return emb_kernel


def kernel(x, lut):
    NB, T = x.shape
    B = NB * T
    lutp = jnp.pad(lut, ((0, 0), (0, W - D)))
    out = _make_emb_kernel(B)(x.reshape(B).astype(jnp.int32), lutp)
    return out[:, :D].reshape(NB, T, D)
